# Initial kernel scaffold; baseline (speedup 1.0000x reference)
import jax
import jax.numpy as jnp
from jax.experimental import pallas as pl


def kernel(x, edge_index, Wl0, bl0, Wr0, Wl1, bl1, Wr1, Wl2, bl2, Wr2):
    raise NotImplementedError("write your pallas kernel here")


# speedup vs baseline: 3.3563x; 3.3563x over previous
import functools

import jax
import jax.numpy as jnp
from jax import lax
from jax.experimental import pallas as pl
from jax.experimental.pallas import tpu as pltpu
from jax.experimental.pallas import tpu_sc as plsc

N = 10000
NP = 10112
E = 160000
K = 128
NC, NS = 2, 16
NW = NC * NS
E_PAD = ((E + NW * K - 1) // (NW * K)) * (NW * K)
EPW = E_PAD // NW
STEPS = EPW // K
RPW = NP // NS

@functools.lru_cache(maxsize=None)
def _make_sc_segsum(d):
    @functools.partial(
        pl.kernel,
        mesh=plsc.VectorSubcoreMesh(core_axis_name="c", subcore_axis_name="s"),
        compiler_params=pltpu.CompilerParams(use_tc_tiling_on_sc=False),
        out_type=jax.ShapeDtypeStruct((NC, NP, d), jnp.float32),
        scratch_types=[
            pltpu.VMEM((K,), jnp.int32),
            pltpu.VMEM((K,), jnp.int32),
            pltpu.VMEM((K, d), jnp.float32),
            pltpu.VMEM_SHARED((NP, d), jnp.float32),
            pltpu.SemaphoreType.DMA,
        ],
    )
    def segsum(y_hbm, src_hbm, dst_hbm, zeros_hbm, out_hbm,
               sidx, didx, rows, accum, sem):
        c = lax.axis_index("c")
        s = lax.axis_index("s")
        wid = s * NC + c
        rbase = s * RPW
        pltpu.sync_copy(zeros_hbm.at[pl.ds(rbase, RPW)],
                        accum.at[pl.ds(rbase, RPW)])
        plsc.subcore_barrier()

        eoff = wid * EPW

        @pl.loop(0, STEPS)
        def _(t):
            b = eoff + t * K
            pltpu.sync_copy(src_hbm.at[pl.ds(b, K)], sidx)
            pltpu.sync_copy(dst_hbm.at[pl.ds(b, K)], didx)
            pltpu.async_copy(y_hbm.at[sidx], rows, sem).wait()
            pltpu.sync_copy(rows, accum.at[didx], add=True)

        plsc.subcore_barrier()
        pltpu.sync_copy(accum.at[pl.ds(rbase, RPW)],
                        out_hbm.at[c, pl.ds(rbase, RPW)])

    return segsum


R = 1000
_GRID = N // R


def _tc0_body(x_ref, wl_ref, wr_ref, xl_ref, xr_ref):
    x = x_ref[...]
    y = jnp.dot(x, wl_ref[...], preferred_element_type=jnp.float32)
    col = lax.broadcasted_iota(jnp.int32, (R, 144), 1)
    xl_ref[...] = y + jnp.where(col == 128, 1.0, 0.0)
    xr_ref[...] = jnp.dot(x, wr_ref[...], preferred_element_type=jnp.float32)


def _tc0(x, wl_ext, wr_t):
    return pl.pallas_call(
        _tc0_body,
        grid=(_GRID,),
        in_specs=[
            pl.BlockSpec((R, 256), lambda i: (i, 0)),
            pl.BlockSpec((256, 144), lambda i: (0, 0)),
            pl.BlockSpec((256, 128), lambda i: (0, 0)),
        ],
        out_specs=[
            pl.BlockSpec((R, 144), lambda i: (i, 0)),
            pl.BlockSpec((R, 128), lambda i: (i, 0)),
        ],
        out_shape=[
            jax.ShapeDtypeStruct((N, 144), jnp.float32),
            jax.ShapeDtypeStruct((N, 128), jnp.float32),
        ],
    )(x, wl_ext, wr_t)


def _fuse1_body(p0_ref, p1_ref, xr_ref, b_ref, wl_ref, wr_ref,
                xl1_ref, xr1_ref, invd_ref):
    p0 = p0_ref[0]
    p1 = p1_ref[0]
    ssum = p0[:, :128] + p1[:, :128]
    deg = p0[:, 128:129] + p1[:, 128:129]
    invd = 1.0 / jnp.maximum(deg, 1.0)
    h = jnp.maximum(ssum * invd + b_ref[...] + xr_ref[...], 0.0)
    xl1_ref[...] = jnp.dot(h, wl_ref[...], preferred_element_type=jnp.float32)
    xr1_ref[...] = jnp.dot(h, wr_ref[...], preferred_element_type=jnp.float32)
    invd_ref[...] = invd


def _fuse1(parts, xr0, bl0, wl1_t, wr1_t):
    return pl.pallas_call(
        _fuse1_body,
        grid=(_GRID,),
        in_specs=[
            pl.BlockSpec((1, R, 144), lambda i: (0, i, 0)),
            pl.BlockSpec((1, R, 144), lambda i: (1, i, 0)),
            pl.BlockSpec((R, 128), lambda i: (i, 0)),
            pl.BlockSpec((1, 128), lambda i: (0, 0)),
            pl.BlockSpec((128, 128), lambda i: (0, 0)),
            pl.BlockSpec((128, 128), lambda i: (0, 0)),
        ],
        out_specs=[
            pl.BlockSpec((R, 128), lambda i: (i, 0)),
            pl.BlockSpec((R, 128), lambda i: (i, 0)),
            pl.BlockSpec((R, 1), lambda i: (i, 0)),
        ],
        out_shape=[
            jax.ShapeDtypeStruct((N, 128), jnp.float32),
            jax.ShapeDtypeStruct((N, 128), jnp.float32),
            jax.ShapeDtypeStruct((N, 1), jnp.float32),
        ],
    )(parts, parts, xr0, bl0, wl1_t, wr1_t)


def _fuse2_body(p0_ref, p1_ref, xr_ref, invd_ref, b_ref, wl_ref, wr_ref,
                xl2_ref, xr2_ref):
    ssum = p0_ref[0] + p1_ref[0]
    h = jnp.maximum(ssum * invd_ref[...] + b_ref[...] + xr_ref[...], 0.0)
    xl2_ref[...] = jnp.dot(h, wl_ref[...], preferred_element_type=jnp.float32)
    xr2_ref[...] = jnp.dot(h, wr_ref[...], preferred_element_type=jnp.float32)


def _fuse2(parts, xr1, invd, bl1, wl2_t, wr2_t):
    return pl.pallas_call(
        _fuse2_body,
        grid=(_GRID,),
        in_specs=[
            pl.BlockSpec((1, R, 128), lambda i: (0, i, 0)),
            pl.BlockSpec((1, R, 128), lambda i: (1, i, 0)),
            pl.BlockSpec((R, 128), lambda i: (i, 0)),
            pl.BlockSpec((R, 1), lambda i: (i, 0)),
            pl.BlockSpec((1, 128), lambda i: (0, 0)),
            pl.BlockSpec((128, 64), lambda i: (0, 0)),
            pl.BlockSpec((128, 64), lambda i: (0, 0)),
        ],
        out_specs=[
            pl.BlockSpec((R, 64), lambda i: (i, 0)),
            pl.BlockSpec((R, 64), lambda i: (i, 0)),
        ],
        out_shape=[
            jax.ShapeDtypeStruct((N, 64), jnp.float32),
            jax.ShapeDtypeStruct((N, 64), jnp.float32),
        ],
    )(parts, parts, xr1, invd, bl1, wl2_t, wr2_t)


def _final_body(p0_ref, p1_ref, xr_ref, invd_ref, b_ref, out_ref):
    ssum = p0_ref[0] + p1_ref[0]
    out_ref[...] = ssum * invd_ref[...] + b_ref[...] + xr_ref[...]


def _final(parts, xr2, invd, bl2):
    return pl.pallas_call(
        _final_body,
        grid=(_GRID,),
        in_specs=[
            pl.BlockSpec((1, R, 64), lambda i: (0, i, 0)),
            pl.BlockSpec((1, R, 64), lambda i: (1, i, 0)),
            pl.BlockSpec((R, 64), lambda i: (i, 0)),
            pl.BlockSpec((R, 1), lambda i: (i, 0)),
            pl.BlockSpec((1, 64), lambda i: (0, 0)),
        ],
        out_specs=pl.BlockSpec((R, 64), lambda i: (i, 0)),
        out_shape=jax.ShapeDtypeStruct((N, 64), jnp.float32),
    )(parts, parts, xr2, invd, bl2)


def kernel(x, edge_index, Wl0, bl0, Wr0, Wl1, bl1, Wr1, Wl2, bl2, Wr2):
    src = edge_index[0]
    dst = edge_index[1]
    pad = E_PAD - E
    src_p = jnp.concatenate([src, jnp.zeros((pad,), jnp.int32)])
    dst_p = jnp.concatenate([dst, jnp.full((pad,), N, jnp.int32)])

    wl0_ext = jnp.concatenate([Wl0.T, jnp.zeros((256, 16), jnp.float32)], axis=1)
    z144 = jnp.zeros((NP, 144), jnp.float32)
    z128 = jnp.zeros((NP, 128), jnp.float32)
    z64 = jnp.zeros((NP, 64), jnp.float32)

    xl0, xr0 = _tc0(x, wl0_ext, Wr0.T)
    parts0 = _make_sc_segsum(144)(xl0, src_p, dst_p, z144)
    xl1, xr1, invd = _fuse1(parts0, xr0, bl0.reshape(1, -1), Wl1.T, Wr1.T)
    parts1 = _make_sc_segsum(128)(xl1, src_p, dst_p, z128)
    xl2, xr2 = _fuse2(parts1, xr1, invd, bl1.reshape(1, -1), Wl2.T, Wr2.T)
    parts2 = _make_sc_segsum(64)(xl2, src_p, dst_p, z64)
    return _final(parts2, xr2, invd, bl2.reshape(1, -1))


# speedup vs baseline: 4.1983x; 1.2508x over previous
import functools

import jax
import jax.numpy as jnp
from jax import lax
from jax.experimental import pallas as pl
from jax.experimental.pallas import tpu as pltpu
from jax.experimental.pallas import tpu_sc as plsc

N = 10000
NP = 10112
E = 160000
NC, NS = 2, 16
NW = NC * NS
E_PAD = 163840
EPW = E_PAD // NW
RPW = NP // NS

@functools.lru_cache(maxsize=None)
def _make_sc_segsum(d, k):
    steps = EPW // k

    @functools.partial(
        pl.kernel,
        mesh=plsc.VectorSubcoreMesh(core_axis_name="c", subcore_axis_name="s"),
        compiler_params=pltpu.CompilerParams(use_tc_tiling_on_sc=False),
        out_type=jax.ShapeDtypeStruct((NC, NP, d), jnp.float32),
        scratch_types=[
            pltpu.VMEM((steps, k), jnp.int32),
            pltpu.VMEM((steps, k), jnp.int32),
            pltpu.VMEM((k, d), jnp.float32),
            pltpu.VMEM((k, d), jnp.float32),
            pltpu.VMEM_SHARED((NP, d), jnp.float32),
            pltpu.SemaphoreType.DMA,
            pltpu.SemaphoreType.DMA,
            pltpu.SemaphoreType.DMA,
            pltpu.SemaphoreType.DMA,
        ],
    )
    def segsum(y_hbm, src_hbm, dst_hbm, zeros_hbm, out_hbm,
               sidx, didx, rows0, rows1, accum, gs0, gs1, ss0, ss1):
        c = lax.axis_index("c")
        s = lax.axis_index("s")
        wid = s * NC + c
        rbase = s * RPW
        rows = (rows0, rows1)
        gs = (gs0, gs1)
        ss = (ss0, ss1)

        pltpu.sync_copy(src_hbm.at[wid], sidx)
        pltpu.sync_copy(dst_hbm.at[wid], didx)
        pltpu.sync_copy(zeros_hbm.at[pl.ds(rbase, RPW)],
                        accum.at[pl.ds(rbase, RPW)])
        plsc.subcore_barrier()

        def start_gather(t, b):
            pltpu.async_copy(y_hbm.at[sidx.at[t]], rows[b], gs[b])

        def wait_gather(b):
            pltpu.make_async_copy(y_hbm.at[sidx.at[0]], rows[b], gs[b]).wait()

        def start_scatter(t, b):
            pltpu.async_copy(rows[b], accum.at[didx.at[t]], ss[b], add=True)

        def wait_scatter(b):
            pltpu.make_async_copy(rows[b], accum.at[didx.at[0]], ss[b]).wait()

        start_gather(0, 0)
        start_gather(1, 1)

        @pl.loop(0, (steps - 2) // 2)
        def _(u):
            for b in range(2):
                t = 2 * u + b
                wait_gather(b)
                start_scatter(t, b)
                wait_scatter(b)
                start_gather(t + 2, b)

        for b in range(2):
            t = steps - 2 + b
            wait_gather(b)
            start_scatter(t, b)
            wait_scatter(b)

        plsc.subcore_barrier()
        pltpu.sync_copy(accum.at[pl.ds(rbase, RPW)],
                        out_hbm.at[c, pl.ds(rbase, RPW)])

    return segsum


R = 1000
_GRID = N // R


def _tc0_body(x_ref, wl_ref, wr_ref, xl_ref, xr_ref):
    x = x_ref[...]
    y = jnp.dot(x, wl_ref[...], preferred_element_type=jnp.float32)
    col = lax.broadcasted_iota(jnp.int32, (R, 144), 1)
    xl_ref[...] = y + jnp.where(col == 128, 1.0, 0.0)
    xr_ref[...] = jnp.dot(x, wr_ref[...], preferred_element_type=jnp.float32)


def _tc0(x, wl_ext, wr_t):
    return pl.pallas_call(
        _tc0_body,
        grid=(_GRID,),
        in_specs=[
            pl.BlockSpec((R, 256), lambda i: (i, 0)),
            pl.BlockSpec((256, 144), lambda i: (0, 0)),
            pl.BlockSpec((256, 128), lambda i: (0, 0)),
        ],
        out_specs=[
            pl.BlockSpec((R, 144), lambda i: (i, 0)),
            pl.BlockSpec((R, 128), lambda i: (i, 0)),
        ],
        out_shape=[
            jax.ShapeDtypeStruct((N, 144), jnp.float32),
            jax.ShapeDtypeStruct((N, 128), jnp.float32),
        ],
    )(x, wl_ext, wr_t)


def _fuse1_body(p0_ref, p1_ref, xr_ref, b_ref, wl_ref, wr_ref,
                xl1_ref, xr1_ref, invd_ref):
    p0 = p0_ref[0]
    p1 = p1_ref[0]
    ssum = p0[:, :128] + p1[:, :128]
    deg = p0[:, 128:129] + p1[:, 128:129]
    invd = 1.0 / jnp.maximum(deg, 1.0)
    h = jnp.maximum(ssum * invd + b_ref[...] + xr_ref[...], 0.0)
    xl1_ref[...] = jnp.dot(h, wl_ref[...], preferred_element_type=jnp.float32)
    xr1_ref[...] = jnp.dot(h, wr_ref[...], preferred_element_type=jnp.float32)
    invd_ref[...] = invd


def _fuse1(parts, xr0, bl0, wl1_t, wr1_t):
    return pl.pallas_call(
        _fuse1_body,
        grid=(_GRID,),
        in_specs=[
            pl.BlockSpec((1, R, 144), lambda i: (0, i, 0)),
            pl.BlockSpec((1, R, 144), lambda i: (1, i, 0)),
            pl.BlockSpec((R, 128), lambda i: (i, 0)),
            pl.BlockSpec((1, 128), lambda i: (0, 0)),
            pl.BlockSpec((128, 128), lambda i: (0, 0)),
            pl.BlockSpec((128, 128), lambda i: (0, 0)),
        ],
        out_specs=[
            pl.BlockSpec((R, 128), lambda i: (i, 0)),
            pl.BlockSpec((R, 128), lambda i: (i, 0)),
            pl.BlockSpec((R, 1), lambda i: (i, 0)),
        ],
        out_shape=[
            jax.ShapeDtypeStruct((N, 128), jnp.float32),
            jax.ShapeDtypeStruct((N, 128), jnp.float32),
            jax.ShapeDtypeStruct((N, 1), jnp.float32),
        ],
    )(parts, parts, xr0, bl0, wl1_t, wr1_t)


def _fuse2_body(p0_ref, p1_ref, xr_ref, invd_ref, b_ref, wl_ref, wr_ref,
                xl2_ref, xr2_ref):
    ssum = p0_ref[0] + p1_ref[0]
    h = jnp.maximum(ssum * invd_ref[...] + b_ref[...] + xr_ref[...], 0.0)
    xl2_ref[...] = jnp.dot(h, wl_ref[...], preferred_element_type=jnp.float32)
    xr2_ref[...] = jnp.dot(h, wr_ref[...], preferred_element_type=jnp.float32)


def _fuse2(parts, xr1, invd, bl1, wl2_t, wr2_t):
    return pl.pallas_call(
        _fuse2_body,
        grid=(_GRID,),
        in_specs=[
            pl.BlockSpec((1, R, 128), lambda i: (0, i, 0)),
            pl.BlockSpec((1, R, 128), lambda i: (1, i, 0)),
            pl.BlockSpec((R, 128), lambda i: (i, 0)),
            pl.BlockSpec((R, 1), lambda i: (i, 0)),
            pl.BlockSpec((1, 128), lambda i: (0, 0)),
            pl.BlockSpec((128, 64), lambda i: (0, 0)),
            pl.BlockSpec((128, 64), lambda i: (0, 0)),
        ],
        out_specs=[
            pl.BlockSpec((R, 64), lambda i: (i, 0)),
            pl.BlockSpec((R, 64), lambda i: (i, 0)),
        ],
        out_shape=[
            jax.ShapeDtypeStruct((N, 64), jnp.float32),
            jax.ShapeDtypeStruct((N, 64), jnp.float32),
        ],
    )(parts, parts, xr1, invd, bl1, wl2_t, wr2_t)


def _final_body(p0_ref, p1_ref, xr_ref, invd_ref, b_ref, out_ref):
    ssum = p0_ref[0] + p1_ref[0]
    out_ref[...] = ssum * invd_ref[...] + b_ref[...] + xr_ref[...]


def _final(parts, xr2, invd, bl2):
    return pl.pallas_call(
        _final_body,
        grid=(_GRID,),
        in_specs=[
            pl.BlockSpec((1, R, 64), lambda i: (0, i, 0)),
            pl.BlockSpec((1, R, 64), lambda i: (1, i, 0)),
            pl.BlockSpec((R, 64), lambda i: (i, 0)),
            pl.BlockSpec((R, 1), lambda i: (i, 0)),
            pl.BlockSpec((1, 64), lambda i: (0, 0)),
        ],
        out_specs=pl.BlockSpec((R, 64), lambda i: (i, 0)),
        out_shape=jax.ShapeDtypeStruct((N, 64), jnp.float32),
    )(parts, parts, xr2, invd, bl2)


def kernel(x, edge_index, Wl0, bl0, Wr0, Wl1, bl1, Wr1, Wl2, bl2, Wr2):
    src = edge_index[0]
    dst = edge_index[1]
    pad = E_PAD - E
    src_f = jnp.concatenate([src, jnp.zeros((pad,), jnp.int32)])
    dst_f = jnp.concatenate([dst, jnp.full((pad,), N, jnp.int32)])
    src80 = src_f.reshape(NW, EPW // 80, 80)
    dst80 = dst_f.reshape(NW, EPW // 80, 80)
    src128 = src_f.reshape(NW, EPW // 128, 128)
    dst128 = dst_f.reshape(NW, EPW // 128, 128)

    wl0_ext = jnp.concatenate([Wl0.T, jnp.zeros((256, 16), jnp.float32)], axis=1)
    z144 = jnp.zeros((NP, 144), jnp.float32)
    z128 = jnp.zeros((NP, 128), jnp.float32)
    z64 = jnp.zeros((NP, 64), jnp.float32)

    xl0, xr0 = _tc0(x, wl0_ext, Wr0.T)
    parts0 = _make_sc_segsum(144, 80)(xl0, src80, dst80, z144)
    xl1, xr1, invd = _fuse1(parts0, xr0, bl0.reshape(1, -1), Wl1.T, Wr1.T)
    parts1 = _make_sc_segsum(128, 128)(xl1, src128, dst128, z128)
    xl2, xr2 = _fuse2(parts1, xr1, invd, bl1.reshape(1, -1), Wl2.T, Wr2.T)
    parts2 = _make_sc_segsum(64, 128)(xl2, src128, dst128, z64)
    return _final(parts2, xr2, invd, bl2.reshape(1, -1))


# speedup vs baseline: 4.2049x; 1.0016x over previous
import functools

import jax
import jax.numpy as jnp
from jax import lax
from jax.experimental import pallas as pl
from jax.experimental.pallas import tpu as pltpu
from jax.experimental.pallas import tpu_sc as plsc

N = 10000
NP = 10112
E = 160000
NC, NS = 2, 16
NW = NC * NS
E_PAD = 163840
EPW = E_PAD // NW
RPW = NP // NS

@functools.lru_cache(maxsize=None)
def _make_sc_segsum(d, k):
    steps = EPW // k

    @functools.partial(
        pl.kernel,
        mesh=plsc.VectorSubcoreMesh(core_axis_name="c", subcore_axis_name="s"),
        compiler_params=pltpu.CompilerParams(use_tc_tiling_on_sc=False),
        out_type=jax.ShapeDtypeStruct((NC, NP, d), jnp.float32),
        scratch_types=[
            pltpu.VMEM((steps, k), jnp.int32),
            pltpu.VMEM((steps, k), jnp.int32),
            [pltpu.VMEM((k, d), jnp.float32)] * 4,
            pltpu.VMEM_SHARED((NP, d), jnp.float32),
            [pltpu.SemaphoreType.DMA] * 4,
            [pltpu.SemaphoreType.DMA] * 4,
        ],
    )
    def segsum(y_hbm, src_hbm, dst_hbm, zeros_hbm, out_hbm,
               sidx, didx, rows, accum, gs, ss):
        c = lax.axis_index("c")
        s = lax.axis_index("s")
        wid = s * NC + c
        rbase = s * RPW

        pltpu.sync_copy(src_hbm.at[wid], sidx)
        pltpu.sync_copy(dst_hbm.at[wid], didx)
        pltpu.sync_copy(zeros_hbm.at[pl.ds(rbase, RPW)],
                        accum.at[pl.ds(rbase, RPW)])
        plsc.subcore_barrier()

        def start_gather(t, b):
            pltpu.async_copy(y_hbm.at[sidx.at[t]], rows[b], gs[b])

        def wait_gather(b):
            pltpu.make_async_copy(y_hbm.at[sidx.at[0]], rows[b], gs[b]).wait()

        def start_scatter(t, b):
            pltpu.async_copy(rows[b], accum.at[didx.at[t]], ss[b], add=True)

        def wait_scatter(b):
            pltpu.make_async_copy(rows[b], accum.at[didx.at[0]], ss[b]).wait()

        start_gather(0, 0)
        start_gather(1, 1)
        for t in range(2):
            wait_gather(t % 4)
            start_gather(t + 2, (t + 2) % 4)
            start_scatter(t, t % 4)

        @pl.loop(0, (steps - 4) // 4)
        def _(u):
            for b in range(4):
                t = 4 * u + 2 + b
                bb = (2 + b) % 4
                wait_gather(bb)
                wait_scatter(b)
                start_gather(t + 2, b)
                start_scatter(t, bb)

        for i in range(2):
            t = steps - 2 + i
            b = t % 4
            wait_gather(b)
            start_scatter(t, b)
        for i in range(4):
            wait_scatter((steps - 4 + i) % 4)

        plsc.subcore_barrier()
        pltpu.sync_copy(accum.at[pl.ds(rbase, RPW)],
                        out_hbm.at[c, pl.ds(rbase, RPW)])

    return segsum


R = 1000
_GRID = N // R


def _tc0_body(x_ref, wl_ref, wr_ref, xl_ref, xr_ref):
    x = x_ref[...]
    y = jnp.dot(x, wl_ref[...], preferred_element_type=jnp.float32)
    col = lax.broadcasted_iota(jnp.int32, (R, 144), 1)
    xl_ref[...] = y + jnp.where(col == 128, 1.0, 0.0)
    xr_ref[...] = jnp.dot(x, wr_ref[...], preferred_element_type=jnp.float32)


def _tc0(x, wl_ext, wr_t):
    return pl.pallas_call(
        _tc0_body,
        grid=(_GRID,),
        in_specs=[
            pl.BlockSpec((R, 256), lambda i: (i, 0)),
            pl.BlockSpec((256, 144), lambda i: (0, 0)),
            pl.BlockSpec((256, 128), lambda i: (0, 0)),
        ],
        out_specs=[
            pl.BlockSpec((R, 144), lambda i: (i, 0)),
            pl.BlockSpec((R, 128), lambda i: (i, 0)),
        ],
        out_shape=[
            jax.ShapeDtypeStruct((N, 144), jnp.float32),
            jax.ShapeDtypeStruct((N, 128), jnp.float32),
        ],
    )(x, wl_ext, wr_t)


def _fuse1_body(p0_ref, p1_ref, xr_ref, b_ref, wl_ref, wr_ref,
                xl1_ref, xr1_ref, invd_ref):
    p0 = p0_ref[0]
    p1 = p1_ref[0]
    ssum = p0[:, :128] + p1[:, :128]
    deg = p0[:, 128:129] + p1[:, 128:129]
    invd = 1.0 / jnp.maximum(deg, 1.0)
    h = jnp.maximum(ssum * invd + b_ref[...] + xr_ref[...], 0.0)
    xl1_ref[...] = jnp.dot(h, wl_ref[...], preferred_element_type=jnp.float32)
    xr1_ref[...] = jnp.dot(h, wr_ref[...], preferred_element_type=jnp.float32)
    invd_ref[...] = invd


def _fuse1(parts, xr0, bl0, wl1_t, wr1_t):
    return pl.pallas_call(
        _fuse1_body,
        grid=(_GRID,),
        in_specs=[
            pl.BlockSpec((1, R, 144), lambda i: (0, i, 0)),
            pl.BlockSpec((1, R, 144), lambda i: (1, i, 0)),
            pl.BlockSpec((R, 128), lambda i: (i, 0)),
            pl.BlockSpec((1, 128), lambda i: (0, 0)),
            pl.BlockSpec((128, 128), lambda i: (0, 0)),
            pl.BlockSpec((128, 128), lambda i: (0, 0)),
        ],
        out_specs=[
            pl.BlockSpec((R, 128), lambda i: (i, 0)),
            pl.BlockSpec((R, 128), lambda i: (i, 0)),
            pl.BlockSpec((R, 1), lambda i: (i, 0)),
        ],
        out_shape=[
            jax.ShapeDtypeStruct((N, 128), jnp.float32),
            jax.ShapeDtypeStruct((N, 128), jnp.float32),
            jax.ShapeDtypeStruct((N, 1), jnp.float32),
        ],
    )(parts, parts, xr0, bl0, wl1_t, wr1_t)


def _fuse2_body(p0_ref, p1_ref, xr_ref, invd_ref, b_ref, wl_ref, wr_ref,
                xl2_ref, xr2_ref):
    ssum = p0_ref[0] + p1_ref[0]
    h = jnp.maximum(ssum * invd_ref[...] + b_ref[...] + xr_ref[...], 0.0)
    xl2_ref[...] = jnp.dot(h, wl_ref[...], preferred_element_type=jnp.float32)
    xr2_ref[...] = jnp.dot(h, wr_ref[...], preferred_element_type=jnp.float32)


def _fuse2(parts, xr1, invd, bl1, wl2_t, wr2_t):
    return pl.pallas_call(
        _fuse2_body,
        grid=(_GRID,),
        in_specs=[
            pl.BlockSpec((1, R, 128), lambda i: (0, i, 0)),
            pl.BlockSpec((1, R, 128), lambda i: (1, i, 0)),
            pl.BlockSpec((R, 128), lambda i: (i, 0)),
            pl.BlockSpec((R, 1), lambda i: (i, 0)),
            pl.BlockSpec((1, 128), lambda i: (0, 0)),
            pl.BlockSpec((128, 64), lambda i: (0, 0)),
            pl.BlockSpec((128, 64), lambda i: (0, 0)),
        ],
        out_specs=[
            pl.BlockSpec((R, 64), lambda i: (i, 0)),
            pl.BlockSpec((R, 64), lambda i: (i, 0)),
        ],
        out_shape=[
            jax.ShapeDtypeStruct((N, 64), jnp.float32),
            jax.ShapeDtypeStruct((N, 64), jnp.float32),
        ],
    )(parts, parts, xr1, invd, bl1, wl2_t, wr2_t)


def _final_body(p0_ref, p1_ref, xr_ref, invd_ref, b_ref, out_ref):
    ssum = p0_ref[0] + p1_ref[0]
    out_ref[...] = ssum * invd_ref[...] + b_ref[...] + xr_ref[...]


def _final(parts, xr2, invd, bl2):
    return pl.pallas_call(
        _final_body,
        grid=(_GRID,),
        in_specs=[
            pl.BlockSpec((1, R, 64), lambda i: (0, i, 0)),
            pl.BlockSpec((1, R, 64), lambda i: (1, i, 0)),
            pl.BlockSpec((R, 64), lambda i: (i, 0)),
            pl.BlockSpec((R, 1), lambda i: (i, 0)),
            pl.BlockSpec((1, 64), lambda i: (0, 0)),
        ],
        out_specs=pl.BlockSpec((R, 64), lambda i: (i, 0)),
        out_shape=jax.ShapeDtypeStruct((N, 64), jnp.float32),
    )(parts, parts, xr2, invd, bl2)


def kernel(x, edge_index, Wl0, bl0, Wr0, Wl1, bl1, Wr1, Wl2, bl2, Wr2):
    src = edge_index[0]
    dst = edge_index[1]
    pad = E_PAD - E
    src_f = jnp.concatenate([src, jnp.zeros((pad,), jnp.int32)])
    dst_f = jnp.concatenate([dst, jnp.full((pad,), N, jnp.int32)])
    def _idx(kk):
        return src_f.reshape(NW, EPW // kk, kk), dst_f.reshape(NW, EPW // kk, kk)
    src40, dst40 = _idx(40)
    src64, dst64 = _idx(64)
    src128, dst128 = _idx(128)

    wl0_ext = jnp.concatenate([Wl0.T, jnp.zeros((256, 16), jnp.float32)], axis=1)
    z144 = jnp.zeros((NP, 144), jnp.float32)
    z128 = jnp.zeros((NP, 128), jnp.float32)
    z64 = jnp.zeros((NP, 64), jnp.float32)

    xl0, xr0 = _tc0(x, wl0_ext, Wr0.T)
    parts0 = _make_sc_segsum(144, 40)(xl0, src40, dst40, z144)
    xl1, xr1, invd = _fuse1(parts0, xr0, bl0.reshape(1, -1), Wl1.T, Wr1.T)
    parts1 = _make_sc_segsum(128, 64)(xl1, src64, dst64, z128)
    xl2, xr2 = _fuse2(parts1, xr1, invd, bl1.reshape(1, -1), Wl2.T, Wr2.T)
    parts2 = _make_sc_segsum(64, 128)(xl2, src128, dst128, z64)
    return _final(parts2, xr2, invd, bl2.reshape(1, -1))


# speedup vs baseline: 5.1187x; 1.2173x over previous
import functools

import jax
import jax.numpy as jnp
from jax import lax
from jax.experimental import pallas as pl
from jax.experimental.pallas import tpu as pltpu
from jax.experimental.pallas import tpu_sc as plsc

N = 10000
NP = 10112
E = 160000
NC, NS = 2, 16
NW = NC * NS
E_PAD = 163840
EPW = E_PAD // NW
RPW = NP // NS

@functools.lru_cache(maxsize=None)
def _make_sc_segsum(d, k, dt=jnp.bfloat16):
    steps = EPW // k

    @functools.partial(
        pl.kernel,
        mesh=plsc.VectorSubcoreMesh(core_axis_name="c", subcore_axis_name="s"),
        compiler_params=pltpu.CompilerParams(use_tc_tiling_on_sc=False),
        out_type=jax.ShapeDtypeStruct((NC, NP, d), dt),
        scratch_types=[
            pltpu.VMEM((steps, k), jnp.int32),
            pltpu.VMEM((steps, k), jnp.int32),
            [pltpu.VMEM((k, d), dt)] * 4,
            pltpu.VMEM_SHARED((NP, d), dt),
            [pltpu.SemaphoreType.DMA] * 4,
            [pltpu.SemaphoreType.DMA] * 4,
        ],
    )
    def segsum(y_hbm, src_hbm, dst_hbm, zeros_hbm, out_hbm,
               sidx, didx, rows, accum, gs, ss):
        c = lax.axis_index("c")
        s = lax.axis_index("s")
        wid = s * NC + c
        rbase = s * RPW

        pltpu.sync_copy(src_hbm.at[wid], sidx)
        pltpu.sync_copy(dst_hbm.at[wid], didx)
        pltpu.sync_copy(zeros_hbm.at[pl.ds(rbase, RPW)],
                        accum.at[pl.ds(rbase, RPW)])
        plsc.subcore_barrier()

        def start_gather(t, b):
            pltpu.async_copy(y_hbm.at[sidx.at[t]], rows[b], gs[b])

        def wait_gather(b):
            pltpu.make_async_copy(y_hbm.at[sidx.at[0]], rows[b], gs[b]).wait()

        def start_scatter(t, b):
            pltpu.async_copy(rows[b], accum.at[didx.at[t]], ss[b], add=True)

        def wait_scatter(b):
            pltpu.make_async_copy(rows[b], accum.at[didx.at[0]], ss[b]).wait()

        start_gather(0, 0)
        start_gather(1, 1)
        for t in range(2):
            wait_gather(t % 4)
            start_gather(t + 2, (t + 2) % 4)
            start_scatter(t, t % 4)

        @pl.loop(0, (steps - 4) // 4)
        def _(u):
            for b in range(4):
                t = 4 * u + 2 + b
                bb = (2 + b) % 4
                wait_gather(bb)
                wait_scatter(b)
                start_gather(t + 2, b)
                start_scatter(t, bb)

        for i in range(2):
            t = steps - 2 + i
            b = t % 4
            wait_gather(b)
            start_scatter(t, b)
        for i in range(4):
            wait_scatter((steps - 4 + i) % 4)

        plsc.subcore_barrier()
        pltpu.sync_copy(accum.at[pl.ds(rbase, RPW)],
                        out_hbm.at[c, pl.ds(rbase, RPW)])

    return segsum


R = 1000
_GRID = N // R


def _tc0_body(x_ref, wl_ref, wr_ref, xl_ref, xr_ref):
    x = x_ref[...]
    y = jnp.dot(x, wl_ref[...], preferred_element_type=jnp.float32)
    col = lax.broadcasted_iota(jnp.int32, (R, 160), 1)
    xl_ref[...] = (y + jnp.where(col == 128, 1.0, 0.0)).astype(jnp.bfloat16)
    xr_ref[...] = jnp.dot(x, wr_ref[...], preferred_element_type=jnp.float32)


def _tc0(x, wl_ext, wr_t):
    return pl.pallas_call(
        _tc0_body,
        grid=(_GRID,),
        in_specs=[
            pl.BlockSpec((R, 256), lambda i: (i, 0)),
            pl.BlockSpec((256, 160), lambda i: (0, 0)),
            pl.BlockSpec((256, 128), lambda i: (0, 0)),
        ],
        out_specs=[
            pl.BlockSpec((R, 160), lambda i: (i, 0)),
            pl.BlockSpec((R, 128), lambda i: (i, 0)),
        ],
        out_shape=[
            jax.ShapeDtypeStruct((N, 160), jnp.bfloat16),
            jax.ShapeDtypeStruct((N, 128), jnp.float32),
        ],
    )(x, wl_ext, wr_t)


def _fuse1_body(p0_ref, p1_ref, xr_ref, b_ref, wl_ref, wr_ref,
                xl1_ref, xr1_ref, invd_ref):
    p0 = p0_ref[0].astype(jnp.float32)
    p1 = p1_ref[0].astype(jnp.float32)
    ssum = p0[:, :128] + p1[:, :128]
    deg = p0[:, 128:129] + p1[:, 128:129]
    invd = 1.0 / jnp.maximum(deg, 1.0)
    h = jnp.maximum(ssum * invd + b_ref[...] + xr_ref[...], 0.0)
    xl1_ref[...] = jnp.dot(h, wl_ref[...],
                           preferred_element_type=jnp.float32).astype(jnp.bfloat16)
    xr1_ref[...] = jnp.dot(h, wr_ref[...], preferred_element_type=jnp.float32)
    invd_ref[...] = invd


def _fuse1(parts, xr0, bl0, wl1_t, wr1_t):
    return pl.pallas_call(
        _fuse1_body,
        grid=(_GRID,),
        in_specs=[
            pl.BlockSpec((1, R, 160), lambda i: (0, i, 0)),
            pl.BlockSpec((1, R, 160), lambda i: (1, i, 0)),
            pl.BlockSpec((R, 128), lambda i: (i, 0)),
            pl.BlockSpec((1, 128), lambda i: (0, 0)),
            pl.BlockSpec((128, 128), lambda i: (0, 0)),
            pl.BlockSpec((128, 128), lambda i: (0, 0)),
        ],
        out_specs=[
            pl.BlockSpec((R, 128), lambda i: (i, 0)),
            pl.BlockSpec((R, 128), lambda i: (i, 0)),
            pl.BlockSpec((R, 1), lambda i: (i, 0)),
        ],
        out_shape=[
            jax.ShapeDtypeStruct((N, 128), jnp.bfloat16),
            jax.ShapeDtypeStruct((N, 128), jnp.float32),
            jax.ShapeDtypeStruct((N, 1), jnp.float32),
        ],
    )(parts, parts, xr0, bl0, wl1_t, wr1_t)


def _fuse2_body(p0_ref, p1_ref, xr_ref, invd_ref, b_ref, wl_ref, wr_ref,
                xl2_ref, xr2_ref):
    ssum = p0_ref[0].astype(jnp.float32) + p1_ref[0].astype(jnp.float32)
    h = jnp.maximum(ssum * invd_ref[...] + b_ref[...] + xr_ref[...], 0.0)
    xl2_ref[...] = jnp.dot(h, wl_ref[...],
                           preferred_element_type=jnp.float32).astype(jnp.bfloat16)
    xr2_ref[...] = jnp.dot(h, wr_ref[...], preferred_element_type=jnp.float32)


def _fuse2(parts, xr1, invd, bl1, wl2_t, wr2_t):
    return pl.pallas_call(
        _fuse2_body,
        grid=(_GRID,),
        in_specs=[
            pl.BlockSpec((1, R, 128), lambda i: (0, i, 0)),
            pl.BlockSpec((1, R, 128), lambda i: (1, i, 0)),
            pl.BlockSpec((R, 128), lambda i: (i, 0)),
            pl.BlockSpec((R, 1), lambda i: (i, 0)),
            pl.BlockSpec((1, 128), lambda i: (0, 0)),
            pl.BlockSpec((128, 64), lambda i: (0, 0)),
            pl.BlockSpec((128, 64), lambda i: (0, 0)),
        ],
        out_specs=[
            pl.BlockSpec((R, 64), lambda i: (i, 0)),
            pl.BlockSpec((R, 64), lambda i: (i, 0)),
        ],
        out_shape=[
            jax.ShapeDtypeStruct((N, 64), jnp.bfloat16),
            jax.ShapeDtypeStruct((N, 64), jnp.float32),
        ],
    )(parts, parts, xr1, invd, bl1, wl2_t, wr2_t)


def _final_body(p0_ref, p1_ref, xr_ref, invd_ref, b_ref, out_ref):
    ssum = p0_ref[0].astype(jnp.float32) + p1_ref[0].astype(jnp.float32)
    out_ref[...] = ssum * invd_ref[...] + b_ref[...] + xr_ref[...]


def _final(parts, xr2, invd, bl2):
    return pl.pallas_call(
        _final_body,
        grid=(_GRID,),
        in_specs=[
            pl.BlockSpec((1, R, 64), lambda i: (0, i, 0)),
            pl.BlockSpec((1, R, 64), lambda i: (1, i, 0)),
            pl.BlockSpec((R, 64), lambda i: (i, 0)),
            pl.BlockSpec((R, 1), lambda i: (i, 0)),
            pl.BlockSpec((1, 64), lambda i: (0, 0)),
        ],
        out_specs=pl.BlockSpec((R, 64), lambda i: (i, 0)),
        out_shape=jax.ShapeDtypeStruct((N, 64), jnp.float32),
    )(parts, parts, xr2, invd, bl2)


def kernel(x, edge_index, Wl0, bl0, Wr0, Wl1, bl1, Wr1, Wl2, bl2, Wr2):
    src = edge_index[0]
    dst = edge_index[1]
    pad = E_PAD - E
    src_f = jnp.concatenate([src, jnp.zeros((pad,), jnp.int32)])
    dst_f = jnp.concatenate([dst, jnp.full((pad,), N, jnp.int32)])
    def _idx(kk):
        return src_f.reshape(NW, EPW // kk, kk), dst_f.reshape(NW, EPW // kk, kk)
    src40, dst40 = _idx(40)
    src64, dst64 = _idx(64)
    src128, dst128 = _idx(128)

    wl0_ext = jnp.concatenate([Wl0.T, jnp.zeros((256, 32), jnp.float32)], axis=1)
    z160 = jnp.zeros((NP, 160), jnp.bfloat16)
    z128 = jnp.zeros((NP, 128), jnp.bfloat16)
    z64 = jnp.zeros((NP, 64), jnp.bfloat16)

    xl0, xr0 = _tc0(x, wl0_ext, Wr0.T)
    parts0 = _make_sc_segsum(160, 64)(xl0, src64, dst64, z160)
    xl1, xr1, invd = _fuse1(parts0, xr0, bl0.reshape(1, -1), Wl1.T, Wr1.T)
    parts1 = _make_sc_segsum(128, 128)(xl1, src128, dst128, z128)
    xl2, xr2 = _fuse2(parts1, xr1, invd, bl1.reshape(1, -1), Wl2.T, Wr2.T)
    parts2 = _make_sc_segsum(64, 128)(xl2, src128, dst128, z64)
    return _final(parts2, xr2, invd, bl2.reshape(1, -1))
